# baseline (speedup 1.0000x reference)
import functools

import jax
import jax.numpy as jnp
from jax import lax
from jax.experimental import pallas as pl
from jax.experimental.pallas import tpu as pltpu
from jax.experimental.pallas import tpu_sc as plsc

B = 4096
L = 200
DIM = 64
MAX_LEN = 100000
NW = 32
LPW = B // NW
NSLOT = 4
LANES = 16
SUB = 8
NBT = B // 128

_mesh = plsc.VectorSubcoreMesh(core_axis_name="c", subcore_axis_name="s")


@functools.partial(
    pl.kernel,
    mesh=_mesh,
    compiler_params=pltpu.CompilerParams(needs_layout_passes=False),
    out_type=jax.ShapeDtypeStruct((L, DIM, B), jnp.float32),
    scratch_types=[
        pltpu.VMEM((L, LPW), jnp.int32),
        pltpu.VMEM((NSLOT, LPW), jnp.int32),
        pltpu.VMEM((NSLOT, LPW, 2 * DIM), jnp.float32),
        pltpu.VMEM((NSLOT, DIM, LPW), jnp.float32),
        pltpu.SemaphoreType.DMA,
        pltpu.SemaphoreType.DMA,
        pltpu.SemaphoreType.DMA,
        pltpu.SemaphoreType.DMA,
        pltpu.SemaphoreType.DMA,
        pltpu.SemaphoreType.DMA,
        pltpu.SemaphoreType.DMA,
        pltpu.SemaphoreType.DMA,
    ],
)
def _pe_kernel(xt_hbm, ow_hbm, tab_hbm, out_hbm, idx_all, hidx_v, rows_v, xs_v,
               l0, l1, l2, l3, o0, o1, o2, o3):
    lsem = (l0, l1, l2, l3)
    osem = (o0, o1, o2, o3)
    wid = lax.axis_index("s") * 2 + lax.axis_index("c")

    c0 = wid * LPW

    pltpu.sync_copy(ow_hbm.at[wid], idx_all)

    lane = lax.iota(jnp.int32, LANES)
    diag = [jnp.bitwise_and(lane + d, LANES - 1) for d in range(LANES)]

    def load(l, s):
        for m in range(LPW // LANES):
            sl = pl.ds(m * LANES, LANES)
            hidx_v[s, sl] = jnp.right_shift(idx_all[l, sl], 1)
        pltpu.async_copy(tab_hbm.at[hidx_v.at[s]], rows_v.at[s], lsem[s])
        pltpu.async_copy(xt_hbm.at[l, :, pl.ds(c0, LPW)], xs_v.at[s], lsem[s])

    def wait_loads(s):
        pltpu.make_async_copy(
            tab_hbm.at[hidx_v.at[0]], rows_v.at[s], lsem[s]).wait()
        pltpu.make_async_copy(
            xt_hbm.at[0, :, pl.ds(0, LPW)], xs_v.at[s], lsem[s]).wait()

    def wait_out(s):
        pltpu.make_async_copy(
            xs_v.at[s], out_hbm.at[0, :, pl.ds(0, LPW)], osem[s]).wait()

    for s in range(NSLOT - 1):
        load(s, s)

    def l_group(p, carry):
        lbase = p * NSLOT
        for s in range(NSLOT):
            l = lbase + s
            wait_loads(s)

            def add_body(ib, carry2):
                babs = lane + ib * LANES
                ordv = idx_all[l, pl.ds(ib * LANES, LANES)]
                parv = jnp.left_shift(jnp.bitwise_and(ordv, 1), 6)
                for k in range(DIM // LANES):
                    cabs = [diag[d] + (k * LANES) for d in range(LANES)]
                    vs = [plsc.load_gather(rows_v.at[s], [babs, cabs[d] + parv])
                          for d in range(LANES)]
                    for d in range(LANES):
                        plsc.addupdate_scatter(xs_v.at[s], [cabs[d], babs], vs[d])
                return carry2

            lax.fori_loop(0, LPW // LANES, add_body, 0, unroll=2)
            pltpu.async_copy(
                xs_v.at[s], out_hbm.at[l, :, pl.ds(c0, LPW)], osem[s])

            ln = l + NSLOT - 1
            sn = (s + NSLOT - 1) % NSLOT

            @pl.when(ln < L)
            def _():
                @pl.when(ln >= NSLOT)
                def _():
                    wait_out(sn)

                load(ln, sn)

        return carry

    lax.fori_loop(0, L // NSLOT, l_group, 0)

    for s in range(NSLOT):
        wait_out(s)


def kernel(x, order, pos_enc):
    xv = jnp.transpose(x, (1, 2, 0))
    ow = jnp.transpose(
        order.astype(jnp.int32).reshape(NW, LPW, L), (0, 2, 1))
    tab2 = pos_enc.reshape(MAX_LEN // 2, 2 * DIM)
    outv = _pe_kernel(xv, ow, tab2)
    return jnp.transpose(outv, (2, 0, 1))


# speedup vs baseline: 1.0867x; 1.0867x over previous
import functools

import jax
import jax.numpy as jnp
from jax import lax
from jax.experimental import pallas as pl
from jax.experimental.pallas import tpu as pltpu
from jax.experimental.pallas import tpu_sc as plsc

B = 4096
L = 200
DIM = 64
MAX_LEN = 100000
NW = 32
LPW = B // NW
NSLOT = 4
LANES = 16
SUB = 8
NBT = B // 128

_mesh = plsc.VectorSubcoreMesh(core_axis_name="c", subcore_axis_name="s")


@functools.partial(
    pl.kernel,
    mesh=_mesh,
    compiler_params=pltpu.CompilerParams(needs_layout_passes=False),
    out_type=jax.ShapeDtypeStruct((L, DIM, B), jnp.float32),
    scratch_types=[
        pltpu.VMEM((L, LPW), jnp.int32),
        pltpu.VMEM((NSLOT, LPW), jnp.int32),
        pltpu.VMEM((NSLOT, LPW, 2 * DIM), jnp.float32),
        pltpu.VMEM((NSLOT, DIM, LPW), jnp.float32),
        pltpu.SemaphoreType.DMA,
        pltpu.SemaphoreType.DMA,
        pltpu.SemaphoreType.DMA,
        pltpu.SemaphoreType.DMA,
        pltpu.SemaphoreType.DMA,
        pltpu.SemaphoreType.DMA,
        pltpu.SemaphoreType.DMA,
        pltpu.SemaphoreType.DMA,
    ],
)
def _pe_kernel(xt_hbm, ow_hbm, tab_hbm, out_hbm, idx_all, hidx_v, rows_v, xs_v,
               l0, l1, l2, l3, o0, o1, o2, o3):
    lsem = (l0, l1, l2, l3)
    osem = (o0, o1, o2, o3)
    wid = lax.axis_index("s") * 2 + lax.axis_index("c")

    c0 = wid * LPW

    pltpu.sync_copy(ow_hbm.at[wid], idx_all)

    lane = lax.iota(jnp.int32, LANES)
    diag = [jnp.bitwise_and(lane + d, LANES - 1) for d in range(LANES)]

    def load(l, s):
        for m in range(LPW // LANES):
            sl = pl.ds(m * LANES, LANES)
            hidx_v[s, sl] = jnp.right_shift(idx_all[l, sl], 1)
        pltpu.async_copy(tab_hbm.at[hidx_v.at[s]], rows_v.at[s], lsem[s])
        pltpu.async_copy(xt_hbm.at[l, :, pl.ds(c0, LPW)], xs_v.at[s], lsem[s])

    def wait_loads(s):
        pltpu.make_async_copy(
            tab_hbm.at[hidx_v.at[0]], rows_v.at[s], lsem[s]).wait()
        pltpu.make_async_copy(
            xt_hbm.at[0, :, pl.ds(0, LPW)], xs_v.at[s], lsem[s]).wait()

    def wait_out(s):
        pltpu.make_async_copy(
            xs_v.at[s], out_hbm.at[0, :, pl.ds(0, LPW)], osem[s]).wait()

    for s in range(NSLOT - 1):
        load(s, s)

    def l_group(p, carry):
        lbase = p * NSLOT
        for s in range(NSLOT):
            l = lbase + s
            wait_loads(s)

            def add_body(ib, carry2):
                babs = lane + ib * LANES
                ordv = idx_all[l, pl.ds(ib * LANES, LANES)]
                parv = jnp.left_shift(jnp.bitwise_and(ordv, 1), 6)
                for k in range(DIM // LANES):
                    cabs = [diag[d] + (k * LANES) for d in range(LANES)]
                    vs = [plsc.load_gather(rows_v.at[s], [babs, cabs[d] + parv])
                          for d in range(LANES)]
                    for d in range(LANES):
                        plsc.addupdate_scatter(xs_v.at[s], [cabs[d], babs], vs[d])
                return carry2

            lax.fori_loop(0, LPW // LANES, add_body, 0)
            pltpu.async_copy(
                xs_v.at[s], out_hbm.at[l, :, pl.ds(c0, LPW)], osem[s])

            ln = l + NSLOT - 1
            sn = (s + NSLOT - 1) % NSLOT

            @pl.when(ln < L)
            def _():
                @pl.when(ln >= NSLOT)
                def _():
                    wait_out(sn)

                load(ln, sn)

        return carry

    lax.fori_loop(0, L // NSLOT, l_group, 0)

    for s in range(NSLOT):
        wait_out(s)


def kernel(x, order, pos_enc):
    xv = jnp.transpose(x, (1, 2, 0))
    ow = jnp.transpose(
        order.astype(jnp.int32).reshape(NW, LPW, L), (0, 2, 1))
    tab2 = pos_enc.reshape(MAX_LEN // 2, 2 * DIM)
    outv = _pe_kernel(xv, ow, tab2)
    return jnp.transpose(outv, (2, 0, 1))
